# baseline (speedup 1.0000x reference)
import functools

import jax
import jax.numpy as jnp
from jax import lax
from jax.experimental import pallas as pl
from jax.experimental.pallas import tpu as pltpu
from jax.experimental.pallas import tpu_sc as plsc

N = 100000
B = 10000
K = 16
D = 256
A = 64

BB = 128
GRID = (B + BB - 1) // BB
CP = 10240
NBP = GRID * BB * K
P = CP + NBP

NW = 32
RPW = P // NW
CHUNK = 128
NCH = RPW // CHUNK


def _sc_gather(table, idx_all):
    mesh = plsc.VectorSubcoreMesh(core_axis_name="c", subcore_axis_name="s")

    @functools.partial(
        pl.kernel,
        mesh=mesh,
        out_type=jax.ShapeDtypeStruct((P, D), jnp.float32),
        scratch_types=[
            pltpu.VMEM((RPW,), jnp.int32),
            pltpu.VMEM((CHUNK, D), jnp.float32),
            pltpu.VMEM((CHUNK, D), jnp.float32),
            pltpu.SemaphoreType.DMA,
            pltpu.SemaphoreType.DMA,
        ],
    )
    def gather_kernel(table_hbm, idx_hbm, out_hbm, idx_v, buf0, buf1, sem0, sem1):
        nc = 2
        wid = lax.axis_index("s") * nc + lax.axis_index("c")
        base = wid * RPW
        pltpu.sync_copy(idx_hbm.at[pl.ds(base, RPW)], idx_v)

        def start(c, buf, sem):
            pltpu.async_copy(
                table_hbm.at[idx_v.at[pl.ds(c * CHUNK, CHUNK)]], buf, sem)

        def finish(c, buf, sem):
            pltpu.make_async_copy(
                table_hbm.at[idx_v.at[pl.ds(c * CHUNK, CHUNK)]], buf, sem).wait()
            pltpu.sync_copy(buf, out_hbm.at[pl.ds(base + c * CHUNK, CHUNK)])

        start(0, buf0, sem0)

        def body(i, carry):
            c0 = i * 2
            start(c0 + 1, buf1, sem1)
            finish(c0, buf0, sem0)

            @pl.when(c0 + 2 < NCH)
            def _():
                start(c0 + 2, buf0, sem0)

            finish(c0 + 1, buf1, sem1)
            return carry

        lax.fori_loop(0, NCH // 2, body, 0)

    return gather_kernel(table, idx_all)


def _tc_body(cen_ref, nbs_ref, nbw_ref, wq_ref, wk_ref, wg_ref, bg_ref,
             wo_ref, bo_ref, gamma_ref, beta_ref, out_ref):
    scale = A ** -0.5
    cen = cen_ref[...]
    nb = nbs_ref[...]
    q = jnp.dot(cen, wq_ref[...], preferred_element_type=jnp.float32)
    k = jnp.dot(nb, wk_ref[...], preferred_element_type=jnp.float32)
    kb = k.reshape(BB, K, A)
    s = jnp.sum(kb * q[:, None, :], axis=2) * scale
    s = s + jnp.log(jnp.maximum(nbw_ref[...], 1e-6))
    s = s - jnp.max(s, axis=1, keepdims=True)
    e = jnp.exp(s)
    attn = e / jnp.sum(e, axis=1, keepdims=True)
    nb3 = nb.reshape(BB, K, D)
    ctx = jnp.sum(nb3 * attn[:, :, None], axis=1)
    gate = jax.nn.sigmoid(
        jnp.dot(cen, wg_ref[...], preferred_element_type=jnp.float32) + bg_ref[...])
    ctx = gate * ctx
    wo = wo_ref[...]
    enr = (jnp.dot(cen, wo[:D], preferred_element_type=jnp.float32)
           + jnp.dot(ctx, wo[D:], preferred_element_type=jnp.float32)
           + bo_ref[...])
    x = enr + cen
    mean = jnp.mean(x, axis=1, keepdims=True)
    xc = x - mean
    var = jnp.mean(xc * xc, axis=1, keepdims=True)
    out_ref[...] = gamma_ref[...] * xc * lax.rsqrt(var + 1e-5) + beta_ref[...]


def _tc_compute(g, nbw, wq, wk, wg, bg, wo, bo, gamma, beta):
    bp = GRID * BB
    return pl.pallas_call(
        _tc_body,
        grid=(GRID,),
        in_specs=[
            pl.BlockSpec((BB, D), lambda b: (b, 0)),
            pl.BlockSpec((BB * K, D), lambda b: (b + CP // (BB * K), 0)),
            pl.BlockSpec((BB, K), lambda b: (b, 0)),
            pl.BlockSpec((D, A), lambda b: (0, 0)),
            pl.BlockSpec((D, A), lambda b: (0, 0)),
            pl.BlockSpec((D, D), lambda b: (0, 0)),
            pl.BlockSpec((1, D), lambda b: (0, 0)),
            pl.BlockSpec((2 * D, D), lambda b: (0, 0)),
            pl.BlockSpec((1, D), lambda b: (0, 0)),
            pl.BlockSpec((1, D), lambda b: (0, 0)),
            pl.BlockSpec((1, D), lambda b: (0, 0)),
        ],
        out_specs=pl.BlockSpec((BB, D), lambda b: (b, 0)),
        out_shape=jax.ShapeDtypeStruct((bp, D), jnp.float32),
    )(g, g, nbw, wq, wk, wg, bg, wo, bo, gamma, beta)


def kernel(all_embs, center_idx, nb_idx, nb_weights, Wq, Wk, Wg, bg, Wo, bo,
           gamma, beta):
    ci = center_idx.astype(jnp.int32)
    nbf = nb_idx.reshape(-1).astype(jnp.int32)
    idx_all = jnp.concatenate([
        jnp.pad(ci, (0, CP - B)),
        jnp.pad(nbf, (0, NBP - B * K)),
    ])
    g = _sc_gather(all_embs, idx_all)
    nbw = jnp.pad(nb_weights, ((0, GRID * BB - B), (0, 0)), constant_values=1.0)
    out = _tc_compute(g, nbw, Wq, Wk, Wg, bg.reshape(1, D), Wo,
                      bo.reshape(1, D), gamma.reshape(1, D), beta.reshape(1, D))
    return out[:B]


# speedup vs baseline: 1.1202x; 1.1202x over previous
import functools

import jax
import jax.numpy as jnp
from jax import lax
from jax.experimental import pallas as pl
from jax.experimental.pallas import tpu as pltpu
from jax.experimental.pallas import tpu_sc as plsc

N = 100000
B = 10000
K = 16
D = 256
A = 64

BB = 128
GRID = (B + BB - 1) // BB
BP = GRID * BB
CP = 10240
NBP = K * BP
P = CP + NBP

NW = 32
RPW = P // NW
CHUNK = 128
NCH = RPW // CHUNK
NCHP = 48


def _sc_gather(table, idx_all):
    mesh = plsc.VectorSubcoreMesh(core_axis_name="c", subcore_axis_name="s")

    @functools.partial(
        pl.kernel,
        mesh=mesh,
        out_type=jax.ShapeDtypeStruct((P, D), jnp.float32),
        scratch_types=[
            pltpu.VMEM((NCHP, CHUNK), jnp.int32),
            pltpu.VMEM((CHUNK, D), jnp.float32),
            pltpu.VMEM((CHUNK, D), jnp.float32),
            pltpu.VMEM((CHUNK, D), jnp.float32),
            pltpu.SemaphoreType.DMA,
            pltpu.SemaphoreType.DMA,
            pltpu.SemaphoreType.DMA,
            pltpu.SemaphoreType.DMA,
            pltpu.SemaphoreType.DMA,
            pltpu.SemaphoreType.DMA,
        ],
    )
    def gather_kernel(table_hbm, idx_hbm, out_hbm, idx_v, buf0, buf1, buf2,
                      sg0, sg1, sg2, sw0, sw1, sw2):
        nc = 2
        wid = lax.axis_index("s") * nc + lax.axis_index("c")
        base = wid * RPW
        bufs = (buf0, buf1, buf2)
        sgs = (sg0, sg1, sg2)
        sws = (sw0, sw1, sw2)
        pltpu.sync_copy(idx_hbm.at[pl.ds(wid * NCHP, NCHP)], idx_v)

        def g_start(c, b):
            pltpu.async_copy(table_hbm.at[idx_v.at[c]], bufs[b], sgs[b])

        def g_wait(b):
            pltpu.make_async_copy(
                table_hbm.at[idx_v.at[0]], bufs[b], sgs[b]).wait()

        def w_start(c, b):
            pltpu.async_copy(
                bufs[b], out_hbm.at[pl.ds(base + c * CHUNK, CHUNK)], sws[b])

        def w_wait(b):
            pltpu.make_async_copy(
                bufs[b], out_hbm.at[pl.ds(base, CHUNK)], sws[b]).wait()

        g_start(0, 0)
        g_start(1, 1)

        def body(i, carry):
            for j in range(3):
                c = i * 3 + j
                bn = (j + 2) % 3

                @pl.when(c >= 1)
                def _():
                    w_wait(bn)

                @pl.when(c + 2 < NCH)
                def _():
                    g_start(c + 2, bn)

                g_wait(j)
                w_start(c, j)
            return carry

        lax.fori_loop(0, NCH // 3, body, 0)
        w_wait((NCH - 1) % 3)

    return gather_kernel(table, idx_all)


def _tc_body(*refs):
    cen_ref = refs[0]
    nb_refs = refs[1:1 + K]
    (nbw_ref, wq_ref, wk_ref, wg_ref, bg_ref, wo_ref, bo_ref, gamma_ref,
     beta_ref, out_ref) = refs[1 + K:]
    scale = A ** -0.5
    cen = cen_ref[...]
    q = jnp.dot(cen, wq_ref[...], preferred_element_type=jnp.float32)
    wk = wk_ref[...]
    nbs = [r[...] for r in nb_refs]
    cols = []
    for k in range(K):
        kp = jnp.dot(nbs[k], wk, preferred_element_type=jnp.float32)
        cols.append(jnp.sum(q * kp, axis=1, keepdims=True))
    s = jnp.concatenate(cols, axis=1) * scale
    s = s + jnp.log(jnp.maximum(nbw_ref[...], 1e-6))
    s = s - jnp.max(s, axis=1, keepdims=True)
    e = jnp.exp(s)
    attn = e / jnp.sum(e, axis=1, keepdims=True)
    ctx = attn[:, 0:1] * nbs[0]
    for k in range(1, K):
        ctx = ctx + attn[:, k:k + 1] * nbs[k]
    gate = jax.nn.sigmoid(
        jnp.dot(cen, wg_ref[...], preferred_element_type=jnp.float32)
        + bg_ref[...])
    ctx = gate * ctx
    wo = wo_ref[...]
    enr = (jnp.dot(cen, wo[:D], preferred_element_type=jnp.float32)
           + jnp.dot(ctx, wo[D:], preferred_element_type=jnp.float32)
           + bo_ref[...])
    x = enr + cen
    mean = jnp.mean(x, axis=1, keepdims=True)
    xc = x - mean
    var = jnp.mean(xc * xc, axis=1, keepdims=True)
    out_ref[...] = gamma_ref[...] * xc * lax.rsqrt(var + 1e-5) + beta_ref[...]


def _nb_spec(k):
    off = CP // BB + k * (BP // BB)
    return pl.BlockSpec((BB, D), lambda b, off=off: (b + off, 0))


def _tc_compute(g, nbw, wq, wk, wg, bg, wo, bo, gamma, beta):
    return pl.pallas_call(
        _tc_body,
        grid=(GRID,),
        in_specs=[pl.BlockSpec((BB, D), lambda b: (b, 0))]
        + [_nb_spec(k) for k in range(K)]
        + [
            pl.BlockSpec((BB, K), lambda b: (b, 0)),
            pl.BlockSpec((D, A), lambda b: (0, 0)),
            pl.BlockSpec((D, A), lambda b: (0, 0)),
            pl.BlockSpec((D, D), lambda b: (0, 0)),
            pl.BlockSpec((1, D), lambda b: (0, 0)),
            pl.BlockSpec((2 * D, D), lambda b: (0, 0)),
            pl.BlockSpec((1, D), lambda b: (0, 0)),
            pl.BlockSpec((1, D), lambda b: (0, 0)),
            pl.BlockSpec((1, D), lambda b: (0, 0)),
        ],
        out_specs=pl.BlockSpec((BB, D), lambda b: (b, 0)),
        out_shape=jax.ShapeDtypeStruct((BP, D), jnp.float32),
    )(g, *([g] * K), nbw, wq, wk, wg, bg, wo, bo, gamma, beta)


def kernel(all_embs, center_idx, nb_idx, nb_weights, Wq, Wk, Wg, bg, Wo, bo,
           gamma, beta):
    ci = center_idx.astype(jnp.int32)
    nbt = jnp.pad(nb_idx.astype(jnp.int32).T, ((0, 0), (0, BP - B)))
    idx_all = jnp.concatenate([jnp.pad(ci, (0, CP - B)), nbt.reshape(-1)])
    idx2 = idx_all.reshape(NW, NCH, CHUNK)
    idx2 = jnp.pad(idx2, ((0, 0), (0, NCHP - NCH), (0, 0)))
    g = _sc_gather(all_embs, idx2.reshape(NW * NCHP, CHUNK))
    nbw = jnp.pad(nb_weights, ((0, BP - B), (0, 0)), constant_values=1.0)
    out = _tc_compute(g, nbw, Wq, Wk, Wg, bg.reshape(1, D), Wo,
                      bo.reshape(1, D), gamma.reshape(1, D), beta.reshape(1, D))
    return out[:B]


# speedup vs baseline: 1.1258x; 1.0050x over previous
import functools

import jax
import jax.numpy as jnp
from jax import lax
from jax.experimental import pallas as pl
from jax.experimental.pallas import tpu as pltpu
from jax.experimental.pallas import tpu_sc as plsc

N = 100000
B = 10000
K = 16
D = 256
A = 64

BB = 128
GRID = (B + BB - 1) // BB
BP = GRID * BB
CP = 10240
NBP = K * BP
P = CP + NBP

NW = 32
RPW = P // NW
CHUNK = 128
NCH = RPW // CHUNK
NCHP = 48


def _sc_gather(table, idx_all):
    mesh = plsc.VectorSubcoreMesh(core_axis_name="c", subcore_axis_name="s")

    @functools.partial(
        pl.kernel,
        mesh=mesh,
        out_type=jax.ShapeDtypeStruct((P, D), jnp.float32),
        scratch_types=[
            pltpu.VMEM((RPW,), jnp.int32),
            pltpu.VMEM((CHUNK, D), jnp.float32),
            pltpu.VMEM((CHUNK, D), jnp.float32),
            pltpu.VMEM((CHUNK, D), jnp.float32),
            pltpu.SemaphoreType.DMA,
            pltpu.SemaphoreType.DMA,
            pltpu.SemaphoreType.DMA,
            pltpu.SemaphoreType.DMA,
            pltpu.SemaphoreType.DMA,
            pltpu.SemaphoreType.DMA,
        ],
    )
    def gather_kernel(table_hbm, idx_hbm, out_hbm, idx_v, buf0, buf1, buf2,
                      sg0, sg1, sg2, sw0, sw1, sw2):
        nc = 2
        wid = lax.axis_index("s") * nc + lax.axis_index("c")
        base = wid * RPW
        bufs = (buf0, buf1, buf2)
        sgs = (sg0, sg1, sg2)
        sws = (sw0, sw1, sw2)
        pltpu.sync_copy(idx_hbm.at[pl.ds(base, RPW)], idx_v)

        def g_start(c, b):
            pltpu.async_copy(
                table_hbm.at[idx_v.at[pl.ds(c * CHUNK, CHUNK)]],
                bufs[b], sgs[b])

        def g_wait(b):
            pltpu.make_async_copy(
                table_hbm.at[idx_v.at[pl.ds(0, CHUNK)]], bufs[b],
                sgs[b]).wait()

        def w_start(c, b):
            pltpu.async_copy(
                bufs[b], out_hbm.at[pl.ds(base + c * CHUNK, CHUNK)], sws[b])

        def w_wait(b):
            pltpu.make_async_copy(
                bufs[b], out_hbm.at[pl.ds(base, CHUNK)], sws[b]).wait()

        g_start(0, 0)
        g_start(1, 1)

        def body(i, carry):
            for j in range(3):
                c = i * 3 + j
                bn = (j + 2) % 3

                @pl.when(c >= 1)
                def _():
                    w_wait(bn)

                @pl.when(c + 2 < NCH)
                def _():
                    g_start(c + 2, bn)

                g_wait(j)
                w_start(c, j)
            return carry

        lax.fori_loop(0, NCH // 3, body, 0)
        w_wait((NCH - 1) % 3)

    return gather_kernel(table, idx_all)


def _tc_body(*refs):
    cen_ref = refs[0]
    nb_refs = refs[1:1 + K]
    (nbw_ref, wq_ref, wk_ref, wg_ref, bg_ref, wo_ref, bo_ref, gamma_ref,
     beta_ref, out_ref) = refs[1 + K:]
    scale = A ** -0.5
    cen = cen_ref[...]
    q = jnp.dot(cen, wq_ref[...], preferred_element_type=jnp.float32)
    wk = wk_ref[...]
    nbs = [r[...] for r in nb_refs]
    cols = []
    for k in range(K):
        kp = jnp.dot(nbs[k], wk, preferred_element_type=jnp.float32)
        cols.append(jnp.sum(q * kp, axis=1, keepdims=True))
    s = jnp.concatenate(cols, axis=1) * scale
    s = s + jnp.log(jnp.maximum(nbw_ref[...], 1e-6))
    s = s - jnp.max(s, axis=1, keepdims=True)
    e = jnp.exp(s)
    attn = e / jnp.sum(e, axis=1, keepdims=True)
    ctx = attn[:, 0:1] * nbs[0]
    for k in range(1, K):
        ctx = ctx + attn[:, k:k + 1] * nbs[k]
    gate = jax.nn.sigmoid(
        jnp.dot(cen, wg_ref[...], preferred_element_type=jnp.float32)
        + bg_ref[...])
    ctx = gate * ctx
    wo = wo_ref[...]
    enr = (jnp.dot(cen, wo[:D], preferred_element_type=jnp.float32)
           + jnp.dot(ctx, wo[D:], preferred_element_type=jnp.float32)
           + bo_ref[...])
    x = enr + cen
    mean = jnp.mean(x, axis=1, keepdims=True)
    xc = x - mean
    var = jnp.mean(xc * xc, axis=1, keepdims=True)
    out_ref[...] = gamma_ref[...] * xc * lax.rsqrt(var + 1e-5) + beta_ref[...]


def _nb_spec(k):
    off = CP // BB + k * (BP // BB)
    return pl.BlockSpec((BB, D), lambda b, off=off: (b + off, 0))


def _tc_compute(g, nbw, wq, wk, wg, bg, wo, bo, gamma, beta):
    return pl.pallas_call(
        _tc_body,
        grid=(GRID,),
        in_specs=[pl.BlockSpec((BB, D), lambda b: (b, 0))]
        + [_nb_spec(k) for k in range(K)]
        + [
            pl.BlockSpec((BB, K), lambda b: (b, 0)),
            pl.BlockSpec((D, A), lambda b: (0, 0)),
            pl.BlockSpec((D, A), lambda b: (0, 0)),
            pl.BlockSpec((D, D), lambda b: (0, 0)),
            pl.BlockSpec((1, D), lambda b: (0, 0)),
            pl.BlockSpec((2 * D, D), lambda b: (0, 0)),
            pl.BlockSpec((1, D), lambda b: (0, 0)),
            pl.BlockSpec((1, D), lambda b: (0, 0)),
            pl.BlockSpec((1, D), lambda b: (0, 0)),
        ],
        out_specs=pl.BlockSpec((BB, D), lambda b: (b, 0)),
        out_shape=jax.ShapeDtypeStruct((BP, D), jnp.float32),
    )(g, *([g] * K), nbw, wq, wk, wg, bg, wo, bo, gamma, beta)


def kernel(all_embs, center_idx, nb_idx, nb_weights, Wq, Wk, Wg, bg, Wo, bo,
           gamma, beta):
    ci = center_idx.astype(jnp.int32)
    nbt = jnp.pad(nb_idx.astype(jnp.int32).T, ((0, 0), (0, BP - B)))
    idx_all = jnp.concatenate([jnp.pad(ci, (0, CP - B)), nbt.reshape(-1)])
    g = _sc_gather(all_embs, idx_all)
    nbw = jnp.pad(nb_weights, ((0, BP - B), (0, 0)), constant_values=1.0)
    out = _tc_compute(g, nbw, Wq, Wk, Wg, bg.reshape(1, D), Wo,
                      bo.reshape(1, D), gamma.reshape(1, D), beta.reshape(1, D))
    return out[:B]
